# baseline (speedup 1.0000x reference)
import jax
import jax.numpy as jnp
from jax import lax
from jax.experimental import pallas as pl
from jax.experimental.pallas import tpu as pltpu

R, C = 128, 32768
BC = 16384
NB = C // BC
RH = R // 2
INT_MAX = 2147483647


def _body(x0_ref, x1_ref, out_ref, m_scr, i_scr, zsc, fix_scr, ismem,
          zsem, isem, fsem):
    j = pl.program_id(0)
    liota = lax.broadcasted_iota(jnp.int32, (RH, BC), 1)
    ms, cis = [], []
    for xr in (x0_ref, x1_ref):
        blk = xr[...]
        m = jnp.max(blk, axis=1, keepdims=True)
        cand = jnp.where(blk == m, liota, INT_MAX)
        ci = jnp.min(cand, axis=1, keepdims=True) + j * BC
        ms.append(m)
        cis.append(ci)
    m = jnp.concatenate(ms, axis=0)
    ci = jnp.concatenate(cis, axis=0)

    @pl.when(j == 0)
    def _():
        m_scr[...] = m
        i_scr[...] = ci
        zsc[...] = jnp.zeros((R, BC), jnp.float32)

    @pl.when(j > 0)
    def _():
        upd = m > m_scr[...]
        i_scr[...] = jnp.where(upd, ci, i_scr[...])
        m_scr[...] = jnp.where(upd, m, m_scr[...])

    pltpu.make_async_copy(
        zsc.at[pl.ds(0, RH), :],
        out_ref.at[pl.ds(0, RH), pl.ds(j * BC, BC)],
        zsem,
    ).start()
    pltpu.make_async_copy(
        zsc.at[pl.ds(RH, RH), :],
        out_ref.at[pl.ds(RH, RH), pl.ds(j * BC, BC)],
        zsem,
    ).start()

    @pl.when(j == NB - 1)
    def _():
        pltpu.make_async_copy(i_scr, ismem, isem).start()
        lane = lax.broadcasted_iota(jnp.int32, (R, 128), 1)
        fix_scr[...] = jnp.where(
            lane == i_scr[...] % 128, 1.0, 0.0
        ).astype(jnp.float32)

        def zdrain(_, c):
            pltpu.make_async_copy(
                zsc.at[pl.ds(0, RH), :],
                out_ref.at[pl.ds(0, RH), pl.ds(0, BC)],
                zsem,
            ).wait()
            return c

        lax.fori_loop(0, 2 * NB, zdrain, 0)
        pltpu.make_async_copy(i_scr, ismem, isem).wait()

        def fire(r, c):
            base = (ismem[r, 0] // 128) * 128
            pltpu.make_async_copy(
                fix_scr.at[pl.ds(r, 1), :],
                out_ref.at[pl.ds(r, 1), pl.ds(base, 128)],
                fsem,
            ).start()
            return c

        lax.fori_loop(0, R, fire, 0)

        def fdrain(_, c):
            pltpu.make_async_copy(
                fix_scr.at[pl.ds(0, 1), :],
                out_ref.at[pl.ds(0, 1), pl.ds(0, 128)],
                fsem,
            ).wait()
            return c

        lax.fori_loop(0, R, fdrain, 0)


def kernel(x):
    return pl.pallas_call(
        _body,
        grid=(NB,),
        in_specs=[
            pl.BlockSpec((RH, BC), lambda j: (0, j)),
            pl.BlockSpec((RH, BC), lambda j: (1, j)),
        ],
        out_specs=pl.BlockSpec(memory_space=pl.ANY),
        out_shape=jax.ShapeDtypeStruct((R, C), jnp.float32),
        scratch_shapes=[
            pltpu.VMEM((R, 1), jnp.float32),
            pltpu.VMEM((R, 1), jnp.int32),
            pltpu.VMEM((R, BC), jnp.float32),
            pltpu.VMEM((R, 128), jnp.float32),
            pltpu.SMEM((R, 1), jnp.int32),
            pltpu.SemaphoreType.DMA,
            pltpu.SemaphoreType.DMA,
            pltpu.SemaphoreType.DMA,
        ],
        compiler_params=pltpu.CompilerParams(
            dimension_semantics=("arbitrary",),
        ),
    )(x, x)
